# baseline (speedup 1.0000x reference)
import functools

import jax
import jax.numpy as jnp
from jax import lax
from jax.experimental import pallas as pl
from jax.experimental.pallas import tpu as pltpu
from jax.experimental.pallas import tpu_sc as plsc

_CACHE_FRAMES = 256
_NEW_FRAMES = 64
_TOK = 128
_D = 16 * 64

_NSUB = 16
_FRAMES_PER_SUB = _CACHE_FRAMES // _NSUB
_R = 32
_CPF = _TOK // _R
_JOBS = _FRAMES_PER_SUB * _CPF
_NSLOT = 2


def _sc_store(idx32, kf, vf, kc, vc):
    mesh = plsc.VectorSubcoreMesh(core_axis_name="c", subcore_axis_name="s")

    @functools.partial(
        pl.kernel,
        out_type=(
            jax.ShapeDtypeStruct((_CACHE_FRAMES, _TOK, _D), jnp.float32),
            jax.ShapeDtypeStruct((_CACHE_FRAMES, _TOK, _D), jnp.float32),
        ),
        mesh=mesh,
        scratch_types=[
            pltpu.SMEM((_NEW_FRAMES,), jnp.int32),
            pltpu.SMEM((_CACHE_FRAMES,), jnp.int32),
            pltpu.VMEM((_NEW_FRAMES,), jnp.int32),
            pltpu.VMEM((_NSLOT, _R, _D), jnp.float32),
            pltpu.SemaphoreType.DMA,
            pltpu.SemaphoreType.DMA((_NSLOT,)),
            pltpu.SemaphoreType.DMA((_NSLOT,)),
        ],
    )
    def store(idx_h, kf_h, vf_h, kc_h, vc_h, ok_h, ov_h,
              idx_s, src_s, idx_v, buf, isem, in_sems, out_sems):
        core = lax.axis_index("c")
        sub = lax.axis_index("s")
        pltpu.sync_copy(idx_h, idx_v)

        @pl.loop(0, _NEW_FRAMES // 16)
        def _(g):
            vec = idx_v[pl.ds(g * 16, 16)]
            for t in range(16):
                idx_s[g * 16 + t] = vec[t]

        @pl.loop(0, _CACHE_FRAMES)
        def _(j):
            src_s[j] = -1

        @pl.loop(0, _NEW_FRAMES)
        def _(i):
            src_s[idx_s[i]] = i

        base = sub * _FRAMES_PER_SUB

        def run(new_h, cache_h, out_h):
            def sync_in(g, slot):
                frame = base + g // _CPF
                row0 = (g % _CPF) * _R
                s = src_s[frame]

                @pl.when(s >= 0)
                def _():
                    pltpu.sync_copy(new_h.at[s, pl.ds(row0, _R)], buf.at[slot])

                @pl.when(s < 0)
                def _():
                    pltpu.sync_copy(cache_h.at[frame, pl.ds(row0, _R)],
                                    buf.at[slot])

            def wait_out(slot):
                pltpu.make_async_copy(buf.at[slot],
                                      out_h.at[0, pl.ds(0, _R)],
                                      out_sems.at[slot]).wait()

            @pl.loop(0, _JOBS, step=_NSLOT)
            def _(it):
                for slot in range(_NSLOT):
                    g = it + slot
                    frame = base + g // _CPF
                    row0 = (g % _CPF) * _R

                    @pl.when(g >= _NSLOT)
                    def _():
                        wait_out(slot)

                    sync_in(g, slot)
                    pltpu.async_copy(buf.at[slot],
                                     out_h.at[frame, pl.ds(row0, _R)],
                                     out_sems.at[slot])

            for slot in range(_NSLOT):
                wait_out(slot)

        @pl.when(core == 0)
        def _():
            run(kf_h, kc_h, ok_h)

        @pl.when(core == 1)
        def _():
            run(vf_h, vc_h, ov_h)

    return store(idx32, kf, vf, kc, vc)


def kernel(k, v, idx, k_cache, v_cache):
    idx32 = idx.astype(jnp.int32) % _CACHE_FRAMES
    out_k, out_v = _sc_store(
        idx32,
        k.reshape(_NEW_FRAMES, _TOK, _D),
        v.reshape(_NEW_FRAMES, _TOK, _D),
        k_cache.reshape(_CACHE_FRAMES, _TOK, _D),
        v_cache.reshape(_CACHE_FRAMES, _TOK, _D))
    return out_k.reshape(k_cache.shape), out_v.reshape(v_cache.shape)
